# Initial kernel scaffold; baseline (speedup 1.0000x reference)
import jax
import jax.numpy as jnp
from jax.experimental import pallas as pl


def kernel(x):
    raise NotImplementedError("write your pallas kernel here")


# speedup vs baseline: 16.4035x; 16.4035x over previous
import jax
import jax.numpy as jnp
from jax import lax
from jax.experimental import pallas as pl

_K = 25088
_ROW = 224 * 224
_ROWS_PER_BLOCK = 8


def _tc_body(x_ref, o_ref):
    xb = x_ref[...]
    ab = lax.bitcast_convert_type(xb, jnp.int32) & jnp.int32(0x7FFFFFFF)

    def step(j, t):
        bit = 30 - j
        cand = t | lax.shift_left(jnp.int32(1), bit)
        cnt = jnp.sum((ab >= cand).astype(jnp.int32), axis=1, keepdims=True)
        return jnp.where(cnt >= _K, cand, t)

    t0 = jnp.zeros((xb.shape[0], 1), jnp.int32)
    t = lax.fori_loop(0, 31, step, t0)
    o_ref[...] = jnp.where(ab >= t, xb, jnp.float32(0.0))


def kernel(x):
    B, C, H, W = x.shape
    rows = B * C
    x2 = x.reshape(rows, H * W)
    out = pl.pallas_call(
        _tc_body,
        grid=(rows // _ROWS_PER_BLOCK,),
        in_specs=[pl.BlockSpec((_ROWS_PER_BLOCK, H * W), lambda i: (i, 0))],
        out_specs=pl.BlockSpec((_ROWS_PER_BLOCK, H * W), lambda i: (i, 0)),
        out_shape=jax.ShapeDtypeStruct((rows, H * W), x.dtype),
    )(x2)
    return out.reshape(B, C, H, W)
